# Initial kernel scaffold; baseline (speedup 1.0000x reference)
import jax
import jax.numpy as jnp
from jax.experimental import pallas as pl


def kernel(hashes, emb_table, pe, W, b):
    raise NotImplementedError("write your pallas kernel here")


# speedup vs baseline: 3.7525x; 3.7525x over previous
import functools

import jax
import jax.numpy as jnp
from jax import lax
from jax.experimental import pallas as pl
from jax.experimental.pallas import tpu as pltpu
from jax.experimental.pallas import tpu_sc as plsc

NUM_PARTS = 100
KP1 = 10000
OUT_DIMS = 64
BATCH = 16384
VOCAB = NUM_PARTS * KP1
TOKENS = BATCH * NUM_PARTS

NUM_CORES = 2
NUM_SUBCORES = 16
NUM_WORKERS = NUM_CORES * NUM_SUBCORES
GATHER_ROWS = 128
CHUNKS_TOTAL = TOKENS // GATHER_ROWS
CHUNKS_PER_W = CHUNKS_TOTAL // NUM_WORKERS


def _idx_body(h_ref, o_ref):
    part = lax.broadcasted_iota(jnp.int32, o_ref.shape, 1)
    o_ref[...] = h_ref[...] + part * KP1


def _table_body(emb_ref, pe_ref, wt_ref, b_ref, o_ref):
    p = pl.program_id(0)
    x = emb_ref[...] + pe_ref[pl.ds(p, 1), :]
    o_ref[...] = (
        jnp.dot(x, wt_ref[...], preferred_element_type=jnp.float32) + b_ref[...]
    )


def _gather_body(table_hbm, idx_hbm, out_hbm, idx_v, rows_v, sem):
    wid = lax.axis_index("s") * NUM_CORES + lax.axis_index("c")
    base = wid * CHUNKS_PER_W

    def step(g, carry):
        row = base + g
        pltpu.sync_copy(idx_hbm.at[pl.ds(row * GATHER_ROWS, GATHER_ROWS)], idx_v)
        pltpu.async_copy(table_hbm.at[idx_v], rows_v, sem).wait()
        pltpu.sync_copy(rows_v, out_hbm.at[pl.ds(row * GATHER_ROWS, GATHER_ROWS)])
        return carry

    lax.fori_loop(0, CHUNKS_PER_W, step, 0)


def kernel(hashes, emb_table, pe, W, b):
    idx = pl.pallas_call(
        _idx_body,
        out_shape=jax.ShapeDtypeStruct((BATCH, NUM_PARTS), jnp.int32),
    )(hashes.astype(jnp.int32))

    tableW = pl.pallas_call(
        _table_body,
        grid=(NUM_PARTS,),
        in_specs=[
            pl.BlockSpec((KP1, OUT_DIMS), lambda p: (p, 0)),
            pl.BlockSpec((NUM_PARTS, OUT_DIMS), lambda p: (0, 0)),
            pl.BlockSpec((OUT_DIMS, OUT_DIMS), lambda p: (0, 0)),
            pl.BlockSpec((1, OUT_DIMS), lambda p: (0, 0)),
        ],
        out_specs=pl.BlockSpec((KP1, OUT_DIMS), lambda p: (p, 0)),
        out_shape=jax.ShapeDtypeStruct((VOCAB, OUT_DIMS), jnp.float32),
    )(emb_table, pe, W.T, b.reshape(1, OUT_DIMS))

    mesh = plsc.VectorSubcoreMesh(core_axis_name="c", subcore_axis_name="s")
    gather = functools.partial(
        pl.kernel,
        mesh=mesh,
        out_type=jax.ShapeDtypeStruct((TOKENS, OUT_DIMS), jnp.float32),
        scratch_types=[
            pltpu.VMEM((GATHER_ROWS,), jnp.int32),
            pltpu.VMEM((GATHER_ROWS, OUT_DIMS), jnp.float32),
            pltpu.SemaphoreType.DMA,
        ],
        compiler_params=pltpu.CompilerParams(use_tc_tiling_on_sc=False),
    )(_gather_body)
    out_flat = gather(tableW, idx.reshape(TOKENS))

    return out_flat.reshape(BATCH, NUM_PARTS, OUT_DIMS)


# speedup vs baseline: 4.5222x; 1.2051x over previous
import functools

import jax
import jax.numpy as jnp
from jax import lax
from jax.experimental import pallas as pl
from jax.experimental.pallas import tpu as pltpu
from jax.experimental.pallas import tpu_sc as plsc

NUM_PARTS = 100
KP1 = 10000
OUT_DIMS = 64
BATCH = 16384
VOCAB = NUM_PARTS * KP1
TOKENS = BATCH * NUM_PARTS

NUM_CORES = 2
NUM_SUBCORES = 16
NUM_WORKERS = NUM_CORES * NUM_SUBCORES
GATHER_ROWS = 128
GROUP = 512
GATHERS_PER_GROUP = GROUP // GATHER_ROWS
ROWS_PER_W = TOKENS // NUM_WORKERS
NG = ROWS_PER_W // GROUP


def _idx_body(h_ref, o_ref):
    part = lax.broadcasted_iota(jnp.int32, o_ref.shape, 1)
    o_ref[...] = h_ref[...] + part * KP1


def _table_body(emb_ref, pe_ref, wt_ref, b_ref, o_ref):
    p = pl.program_id(0)
    x = emb_ref[...] + pe_ref[pl.ds(p, 1), :]
    o_ref[...] = (
        jnp.dot(x, wt_ref[...], preferred_element_type=jnp.float32) + b_ref[...]
    )


def _gather_body(table_hbm, idx_hbm, out_hbm,
                 idx0, idx1, rows0, rows1,
                 isem0, isem1, gsem0, gsem1, ssem0, ssem1):
    idxb = (idx0, idx1)
    rows = (rows0, rows1)
    isem = (isem0, isem1)
    gsem = (gsem0, gsem1)
    ssem = (ssem0, ssem1)
    wid = lax.axis_index("s") * NUM_CORES + lax.axis_index("c")
    base = wid * ROWS_PER_W

    def idx_load(g, slot):
        return pltpu.async_copy(
            idx_hbm.at[pl.ds(base + g * GROUP, GROUP)], idxb[slot], isem[slot])

    def wait_idx(slot):
        pltpu.make_async_copy(
            idx_hbm.at[pl.ds(base, GROUP)], idxb[slot], isem[slot]).wait()

    def fire_gathers(slot):
        for j in range(GATHERS_PER_GROUP):
            pltpu.async_copy(
                table_hbm.at[idxb[slot].at[pl.ds(j * GATHER_ROWS, GATHER_ROWS)]],
                rows[slot].at[pl.ds(j * GATHER_ROWS, GATHER_ROWS)],
                gsem[slot])

    def wait_gathers(slot):
        pltpu.make_async_copy(
            table_hbm.at[pl.ds(0, GROUP)], rows[slot], gsem[slot]).wait()

    def store(g, slot):
        pltpu.async_copy(
            rows[slot], out_hbm.at[pl.ds(base + g * GROUP, GROUP)], ssem[slot])

    def wait_store(slot):
        pltpu.make_async_copy(
            rows[slot], out_hbm.at[pl.ds(base, GROUP)], ssem[slot]).wait()

    idx_load(0, 0).wait()
    fire_gathers(0)
    idx_load(1, 1)

    def body(i, carry):
        g0 = i * 2
        for b in (0, 1):
            g = g0 + b
            cur, nxt = b, 1 - b
            wait_gathers(cur)

            @pl.when(g + 2 < NG)
            def _():
                idx_load(g + 2, cur)

            @pl.when(g + 1 < NG)
            def _():
                @pl.when(g >= 1)
                def _():
                    wait_store(nxt)
                wait_idx(nxt)
                fire_gathers(nxt)

            store(g, cur)
        return carry

    lax.fori_loop(0, NG // 2, body, 0)
    wait_store(0)
    wait_store(1)


def kernel(hashes, emb_table, pe, W, b):
    idx = pl.pallas_call(
        _idx_body,
        out_shape=jax.ShapeDtypeStruct((BATCH, NUM_PARTS), jnp.int32),
    )(hashes.astype(jnp.int32))

    tableW = pl.pallas_call(
        _table_body,
        grid=(NUM_PARTS,),
        in_specs=[
            pl.BlockSpec((KP1, OUT_DIMS), lambda p: (p, 0)),
            pl.BlockSpec((NUM_PARTS, OUT_DIMS), lambda p: (0, 0)),
            pl.BlockSpec((OUT_DIMS, OUT_DIMS), lambda p: (0, 0)),
            pl.BlockSpec((1, OUT_DIMS), lambda p: (0, 0)),
        ],
        out_specs=pl.BlockSpec((KP1, OUT_DIMS), lambda p: (p, 0)),
        out_shape=jax.ShapeDtypeStruct((VOCAB, OUT_DIMS), jnp.float32),
    )(emb_table, pe, W.T, b.reshape(1, OUT_DIMS))

    mesh = plsc.VectorSubcoreMesh(core_axis_name="c", subcore_axis_name="s")
    gather = functools.partial(
        pl.kernel,
        mesh=mesh,
        out_type=jax.ShapeDtypeStruct((TOKENS, OUT_DIMS), jnp.float32),
        scratch_types=[
            pltpu.VMEM((GROUP,), jnp.int32),
            pltpu.VMEM((GROUP,), jnp.int32),
            pltpu.VMEM((GROUP, OUT_DIMS), jnp.float32),
            pltpu.VMEM((GROUP, OUT_DIMS), jnp.float32),
            pltpu.SemaphoreType.DMA,
            pltpu.SemaphoreType.DMA,
            pltpu.SemaphoreType.DMA,
            pltpu.SemaphoreType.DMA,
            pltpu.SemaphoreType.DMA,
            pltpu.SemaphoreType.DMA,
        ],
        compiler_params=pltpu.CompilerParams(use_tc_tiling_on_sc=False),
    )(_gather_body)
    out_flat = gather(tableW, idx.reshape(TOKENS))

    return out_flat.reshape(BATCH, NUM_PARTS, OUT_DIMS)
